# baseline (speedup 1.0000x reference)
import functools

import jax
import jax.numpy as jnp
from jax import lax
from jax.experimental import pallas as pl
from jax.experimental.pallas import tpu as pltpu
from jax.experimental.pallas import tpu_sc as plsc

_LAMB_COORD = 5.0
_LAMB_NOOBJ = 0.5
_B, _C, _S2 = 2048, 30, 49
_ROW = _C * _S2
_NW = 32
_RPW = _B // _NW
_CHUNK = 8
_XY = (0, 1, 5, 6)
_WH = (2, 3, 7, 8)
_CONF = (4, 9)
_CLS = tuple(range(10, 30))
_STARTS = (0, 16, 32, 33)


def _sqrt16(x):
    x = jnp.maximum(x, 1e-12)
    i = plsc.bitcast(x, jnp.int32)
    i = jnp.int32(0x5F3759DF) - lax.shift_right_logical(i, 1)
    r = plsc.bitcast(i, jnp.float32)
    for _ in range(3):
        r = r * (1.5 - 0.5 * x * r * r)
    return x * r


def _sc_body(p_hbm, g_hbm, out_hbm, p_buf, g_buf, acc_ref):
    cid = lax.axis_index("c")
    sid = lax.axis_index("s")
    wid = sid * 2 + cid
    base = wid * _RPW
    lane = lax.iota(jnp.int32, 16)
    sel3 = jnp.where(lane == 15, 1.0, 0.0).astype(jnp.float32)
    zero = jnp.zeros((16,), jnp.float32)

    def row_body(r, acc):
        for s in _STARTS:
            g_c4 = g_buf[r, pl.ds(4 * _S2 + s, 16)]
            m = jnp.where(g_c4 == 1.0, 1.0, 0.0).astype(jnp.float32)
            s_xy = zero
            for c in _XY:
                pv = p_buf[r, pl.ds(c * _S2 + s, 16)]
                gv = g_buf[r, pl.ds(c * _S2 + s, 16)]
                d = pv - gv
                s_xy = s_xy + d * d
            s_wh = zero
            for c in _WH:
                pv = p_buf[r, pl.ds(c * _S2 + s, 16)]
                gv = g_buf[r, pl.ds(c * _S2 + s, 16)]
                s_wh = s_wh + pv + gv - 2.0 * _sqrt16(pv * gv)
            p_c4 = p_buf[r, pl.ds(4 * _S2 + s, 16)]
            d4 = p_c4 - g_c4
            p_c9 = p_buf[r, pl.ds(9 * _S2 + s, 16)]
            g_c9 = g_buf[r, pl.ds(9 * _S2 + s, 16)]
            d9 = p_c9 - g_c9
            s_conf = d4 * d4 + d9 * d9
            s_cls = zero
            for c in _CLS:
                pv = p_buf[r, pl.ds(c * _S2 + s, 16)]
                gv = g_buf[r, pl.ds(c * _S2 + s, 16)]
                d = pv - gv
                s_cls = s_cls + d * d
            contrib = (m * (_LAMB_COORD * (s_xy + s_wh) + s_cls
                            + (1.0 - _LAMB_NOOBJ) * s_conf)
                       + _LAMB_NOOBJ * s_conf)
            if s == 33:
                contrib = contrib * sel3
            acc = acc + contrib
        return acc

    def chunk_body(ci, acc):
        row0 = base + ci * _CHUNK
        pltpu.sync_copy(p_hbm.at[pl.ds(row0, _CHUNK)], p_buf)
        pltpu.sync_copy(g_hbm.at[pl.ds(row0, _CHUNK)], g_buf)
        return lax.fori_loop(0, _CHUNK, row_body, acc, unroll=False)

    acc = lax.fori_loop(0, _RPW // _CHUNK, chunk_body,
                        jnp.zeros((16,), jnp.float32), unroll=False)
    acc_ref[...] = acc
    pltpu.sync_copy(acc_ref, out_hbm.at[wid])


@jax.jit
def _sc_loss(p2, g2):
    mesh = plsc.VectorSubcoreMesh(core_axis_name="c", subcore_axis_name="s")
    run = pl.kernel(
        _sc_body,
        out_type=jax.ShapeDtypeStruct((_NW, 16), jnp.float32),
        mesh=mesh,
        scratch_types=[
            pltpu.VMEM((_CHUNK, _ROW), jnp.float32),
            pltpu.VMEM((_CHUNK, _ROW), jnp.float32),
            pltpu.VMEM((16,), jnp.float32),
        ],
        compiler_params=pltpu.CompilerParams(needs_layout_passes=False),
    )
    return run(p2, g2)


def kernel(pred, gt):
    b = pred.shape[0]
    p2 = pred.reshape(_B, _ROW)
    g2 = gt.reshape(_B, _ROW)
    partials = _sc_loss(p2, g2)
    return jnp.sum(partials) / b


# speedup vs baseline: 3.8853x; 3.8853x over previous
import jax
import jax.numpy as jnp
from jax import lax
from jax.experimental import pallas as pl
from jax.experimental.pallas import tpu as pltpu
from jax.experimental.pallas import tpu_sc as plsc

_LAMB_COORD = 5.0
_LAMB_NOOBJ = 0.5
_B, _C, _S = 2048, 30, 7
_S2 = _S * _S
_NBT = _B // 128
_NTASK = _S2 * _NBT
_NW = 32
_ITERS = 26
_XY = ((0, 0), (0, 1), (0, 5), (0, 6))
_WH = ((0, 2), (0, 3), (0, 7), (1, 0))
_CONF = ((0, 4), (1, 1))
_CLS = tuple((ct, sub) for ct in (1, 2, 3) for sub in range(8)
             if 10 <= 8 * ct + sub < 30)


def _sqrt16(x):
    x = jnp.maximum(x, 1e-12)
    i = plsc.bitcast(x, jnp.int32)
    i = jnp.int32(0x5F3759DF) - lax.shift_right_logical(i, 1)
    r = plsc.bitcast(i, jnp.float32)
    for _ in range(3):
        r = r * (1.5 - 0.5 * x * r * r)
    return x * r


def _sc_body(p_hbm, g_hbm, out_hbm,
             p_a, g_a, p_b, g_b, acc_ref,
             sem_pa, sem_ga, sem_pb, sem_gb):
    cid = lax.axis_index("c")
    sid = lax.axis_index("s")
    w = sid * 2 + cid

    def issue(t, p_dst, g_dst, sp, sg):
        t = jnp.minimum(t, _NTASK - 1)
        cell = lax.shift_right_logical(t, 4)
        b0 = lax.bitwise_and(t, _NBT - 1) * 128
        for ct in range(4):
            ns = 8 if ct < 3 else _C - 24
            src = p_hbm.at[cell, pl.ds(8 * ct, ns), pl.ds(b0, 128)]
            pltpu.async_copy(src, p_dst.at[ct, pl.ds(0, ns)], sp)
            src = g_hbm.at[cell, pl.ds(8 * ct, ns), pl.ds(b0, 128)]
            pltpu.async_copy(src, g_dst.at[ct, pl.ds(0, ns)], sg)

    def drain(p_dst, g_dst, sp, sg):
        for ct in range(4):
            ns = 8 if ct < 3 else _C - 24
            dummy = p_hbm.at[0, pl.ds(0, ns), pl.ds(0, 128)]
            pltpu.make_async_copy(dummy, p_dst.at[ct, pl.ds(0, ns)], sp).wait()
            pltpu.make_async_copy(dummy, g_dst.at[ct, pl.ds(0, ns)], sg).wait()

    def compute(t, p_buf, g_buf, acc):
        vf = jnp.where(t < _NTASK, 1.0, 0.0).astype(jnp.float32)
        vh = vf * _LAMB_NOOBJ

        def chunk(k, acc):
            o = k * 16
            mg = g_buf[0, 4, pl.ds(o, 16)]
            m = jnp.where(mg == 1.0, vf, 0.0)
            s_xy = None
            for ct, sub in _XY:
                d = p_buf[ct, sub, pl.ds(o, 16)] - g_buf[ct, sub, pl.ds(o, 16)]
                d = d * d
                s_xy = d if s_xy is None else s_xy + d
            s_wh = None
            for ct, sub in _WH:
                pv = p_buf[ct, sub, pl.ds(o, 16)]
                gv = g_buf[ct, sub, pl.ds(o, 16)]
                d = pv + gv - 2.0 * _sqrt16(pv * gv)
                s_wh = d if s_wh is None else s_wh + d
            s_conf = None
            for ct, sub in _CONF:
                d = p_buf[ct, sub, pl.ds(o, 16)] - g_buf[ct, sub, pl.ds(o, 16)]
                d = d * d
                s_conf = d if s_conf is None else s_conf + d
            s_cls = None
            for ct, sub in _CLS:
                d = p_buf[ct, sub, pl.ds(o, 16)] - g_buf[ct, sub, pl.ds(o, 16)]
                d = d * d
                s_cls = d if s_cls is None else s_cls + d
            return acc + (m * (_LAMB_COORD * (s_xy + s_wh) + s_cls
                               + (1.0 - _LAMB_NOOBJ) * s_conf)
                          + vh * s_conf)

        return lax.fori_loop(0, 8, chunk, acc, unroll=False)

    issue(w, p_a, g_a, sem_pa, sem_ga)

    def pair_body(jj, acc):
        t0 = w + _NW * 2 * jj
        issue(t0 + _NW, p_b, g_b, sem_pb, sem_gb)
        drain(p_a, g_a, sem_pa, sem_ga)
        acc = compute(t0, p_a, g_a, acc)
        issue(t0 + 2 * _NW, p_a, g_a, sem_pa, sem_ga)
        drain(p_b, g_b, sem_pb, sem_gb)
        acc = compute(t0 + _NW, p_b, g_b, acc)
        return acc

    acc = lax.fori_loop(0, _ITERS // 2, pair_body,
                        jnp.zeros((16,), jnp.float32), unroll=False)
    drain(p_a, g_a, sem_pa, sem_ga)
    acc_ref[...] = acc
    pltpu.sync_copy(acc_ref, out_hbm.at[pl.ds(w * 16, 16)])


@jax.jit
def _sc_loss(pt, gtt):
    mesh = plsc.VectorSubcoreMesh(core_axis_name="c", subcore_axis_name="s")
    run = pl.kernel(
        _sc_body,
        out_type=jax.ShapeDtypeStruct((_NW * 16,), jnp.float32),
        mesh=mesh,
        scratch_types=[
            pltpu.VMEM((4, 8, 128), jnp.float32),
            pltpu.VMEM((4, 8, 128), jnp.float32),
            pltpu.VMEM((4, 8, 128), jnp.float32),
            pltpu.VMEM((4, 8, 128), jnp.float32),
            pltpu.VMEM((16,), jnp.float32),
            pltpu.SemaphoreType.DMA,
            pltpu.SemaphoreType.DMA,
            pltpu.SemaphoreType.DMA,
            pltpu.SemaphoreType.DMA,
        ],
        compiler_params=pltpu.CompilerParams(
            needs_layout_passes=False, use_tc_tiling_on_sc=True),
    )
    return run(pt, gtt)


def kernel(pred, gt):
    b = pred.shape[0]
    pt = jnp.transpose(pred, (2, 3, 1, 0)).reshape(_S2, _C, _B)
    gtt = jnp.transpose(gt, (2, 3, 1, 0)).reshape(_S2, _C, _B)
    partials = _sc_loss(pt, gtt)
    return jnp.sum(partials) / b
